# baseline (speedup 1.0000x reference)
import jax
import jax.numpy as jnp
from jax import lax
from jax.experimental import pallas as pl
from jax.experimental.pallas import tpu as pltpu
from jax.experimental.pallas import tpu_sc as plsc

_CYCLE = 168
_LEN = 336
_CH = 256
_B = 1024
_NC = 2
_NS = 16
_NW = _NC * _NS
_BPW = _B // _NW


def _sc_body(idx_hbm, data_hbm, out_hbm, idx_v, dd_v, ddd_sh, sem, stage_sem):
    cid = lax.axis_index("c")
    sid = lax.axis_index("s")
    wid = sid * _NC + cid
    base = wid * _BPW
    stage = [
        pltpu.async_copy(idx_hbm.at[pl.ds(base, _BPW)], idx_v, stage_sem),
        pltpu.async_copy(data_hbm, dd_v.at[pl.ds(0, _CYCLE)], stage_sem),
        pltpu.async_copy(data_hbm, dd_v.at[pl.ds(_CYCLE, _CYCLE)], stage_sem),
    ]
    for c in stage:
        c.wait()

    @pl.when(sid == 0)
    def _stage_shared():
        shared = [
            pltpu.async_copy(data_hbm, ddd_sh.at[pl.ds(0, _CYCLE)], stage_sem),
            pltpu.async_copy(data_hbm, ddd_sh.at[pl.ds(_CYCLE, _CYCLE)], stage_sem),
            pltpu.async_copy(data_hbm, ddd_sh.at[pl.ds(2 * _CYCLE, _CYCLE)], stage_sem),
        ]
        for c in shared:
            c.wait()

    plsc.subcore_barrier()
    copies = []
    for g in range(_BPW // 16):
        vec = idx_v[pl.ds(g * 16, 16)]
        for j in range(16):
            b = g * 16 + j
            i = vec[j]
            if b % 2 == 0:
                copies.append(pltpu.async_copy(
                    dd_v.at[pl.ds(i, _CYCLE)],
                    out_hbm.at[base + b, pl.ds(0, _CYCLE)], sem))
                copies.append(pltpu.async_copy(
                    dd_v.at[pl.ds(i, _CYCLE)],
                    out_hbm.at[base + b, pl.ds(_CYCLE, _CYCLE)], sem))
            else:
                copies.append(pltpu.async_copy(
                    ddd_sh.at[pl.ds(i, _LEN)], out_hbm.at[base + b], sem))
    for c in copies:
        c.wait()


def kernel(index, length, data):
    start = jnp.mod(index.astype(jnp.int32) + (length - _LEN), _CYCLE)
    start = start.astype(jnp.int32)
    mesh = plsc.VectorSubcoreMesh(core_axis_name="c", subcore_axis_name="s")
    k = pl.kernel(
        _sc_body,
        out_type=jax.ShapeDtypeStruct((_B, _LEN, _CH), jnp.float32),
        mesh=mesh,
        scratch_types=[
            pltpu.VMEM((_BPW,), jnp.int32),
            pltpu.VMEM((2 * _CYCLE, _CH), jnp.float32),
            pltpu.VMEM_SHARED((3 * _CYCLE, _CH), jnp.float32),
            pltpu.SemaphoreType.DMA,
            pltpu.SemaphoreType.DMA,
        ],
        compiler_params=pltpu.CompilerParams(use_tc_tiling_on_sc=False),
    )
    return k(start, data)


# speedup vs baseline: 1.0067x; 1.0067x over previous
import jax
import jax.numpy as jnp
from jax import lax
from jax.experimental import pallas as pl
from jax.experimental.pallas import tpu as pltpu
from jax.experimental.pallas import tpu_sc as plsc

_CYCLE = 168
_LEN = 336
_CH = 256
_B = 1024
_NC = 2
_NS = 16
_NW = _NC * _NS
_BPW = _B // _NW


def _sc_body(idx_hbm, data_hbm, out_hbm, idx_v, dd_v, ddd_sh, sem, stage_sem):
    cid = lax.axis_index("c")
    sid = lax.axis_index("s")
    wid = sid * _NC + cid
    base = wid * _BPW
    stage = [
        pltpu.async_copy(idx_hbm.at[pl.ds(base, _BPW)], idx_v, stage_sem),
        pltpu.async_copy(data_hbm, dd_v.at[pl.ds(0, _CYCLE)], stage_sem),
        pltpu.async_copy(data_hbm, dd_v.at[pl.ds(_CYCLE, _CYCLE)], stage_sem),
    ]
    for c in stage:
        c.wait()

    @pl.when(sid == 0)
    def _stage_shared():
        shared = [
            pltpu.async_copy(data_hbm, ddd_sh.at[pl.ds(0, _CYCLE)], stage_sem),
            pltpu.async_copy(data_hbm, ddd_sh.at[pl.ds(_CYCLE, _CYCLE)], stage_sem),
            pltpu.async_copy(data_hbm, ddd_sh.at[pl.ds(2 * _CYCLE, _CYCLE)], stage_sem),
        ]
        for c in shared:
            c.wait()

    plsc.subcore_barrier()
    copies = []
    for g in range(_BPW // 16):
        vec = idx_v[pl.ds(g * 16, 16)]
        for j in range(16):
            b = g * 16 + j
            i = vec[j]
            if b % 8 < 5:
                copies.append(pltpu.async_copy(
                    dd_v.at[pl.ds(i, _CYCLE)],
                    out_hbm.at[base + b, pl.ds(0, _CYCLE)], sem))
                copies.append(pltpu.async_copy(
                    dd_v.at[pl.ds(i, _CYCLE)],
                    out_hbm.at[base + b, pl.ds(_CYCLE, _CYCLE)], sem))
            else:
                copies.append(pltpu.async_copy(
                    ddd_sh.at[pl.ds(i, _LEN)], out_hbm.at[base + b], sem))
    for c in copies:
        c.wait()


def kernel(index, length, data):
    start = jnp.mod(index.astype(jnp.int32) + (length - _LEN), _CYCLE)
    start = start.astype(jnp.int32)
    mesh = plsc.VectorSubcoreMesh(core_axis_name="c", subcore_axis_name="s")
    k = pl.kernel(
        _sc_body,
        out_type=jax.ShapeDtypeStruct((_B, _LEN, _CH), jnp.float32),
        mesh=mesh,
        scratch_types=[
            pltpu.VMEM((_BPW,), jnp.int32),
            pltpu.VMEM((2 * _CYCLE, _CH), jnp.float32),
            pltpu.VMEM_SHARED((3 * _CYCLE, _CH), jnp.float32),
            pltpu.SemaphoreType.DMA,
            pltpu.SemaphoreType.DMA,
        ],
        compiler_params=pltpu.CompilerParams(use_tc_tiling_on_sc=False),
    )
    return k(start, data)
